# baseline (speedup 1.0000x reference)
import functools

import jax
import jax.numpy as jnp
from jax import lax
from jax.experimental import pallas as pl
from jax.experimental.pallas import tpu as pltpu
from jax.experimental.pallas import tpu_sc as plsc

N_NODES = 10000
N_EDGES = 320000
F_IN = 128
F_HID = 128
F_OUT = 40
F_OUT_PAD = 128

NC = 2
NS = 16
NW = NC * NS
CHUNK = 128
TPC = 80
E_PAD = NW * TPC * CHUNK
NODE_SLOTS = 5
N_PAD = NODE_SLOTS * NS * CHUNK
NB = 4

ROW_BLK = 1000


def _sc_mesh():
    return plsc.VectorSubcoreMesh(
        core_axis_name="c", subcore_axis_name="s", num_cores=NC, num_subcores=NS
    )


def _fill_zero_2d(buf, rows, d):
    z = jnp.zeros((16,), jnp.float32)

    def row(i, carry):
        for j in range(d // 16):
            buf[i, pl.ds(j * 16, 16)] = z
        return carry

    lax.fori_loop(0, rows, row, 0)


def _make_agg(d):
    @functools.partial(
        pl.kernel,
        out_type=jax.ShapeDtypeStruct((NC, N_PAD, d), jnp.float32),
        mesh=_sc_mesh(),
        scratch_types=(
            [pltpu.VMEM((TPC, CHUNK), jnp.int32)] * 2
            + [pltpu.VMEM((CHUNK, d), jnp.float32)] * NB
            + [pltpu.SemaphoreType.DMA] * (2 * NB)
            + [pltpu.VMEM_SHARED((N_PAD, d), jnp.float32)]
        ),
    )
    def agg(table, edges, out, sidx, didx, g0, g1, g2, g3, gs0, gs1, gs2, gs3,
            ss0, ss1, ss2, ss3, acc):
        gbufs = (g0, g1, g2, g3)
        gsems = (gs0, gs1, gs2, gs3)
        ssems = (ss0, ss1, ss2, ss3)
        c = lax.axis_index("c")
        s = lax.axis_index("s")
        tid = c * NS + s

        pltpu.sync_copy(edges.at[0, pl.ds(tid * TPC, TPC)], sidx)
        pltpu.sync_copy(edges.at[1, pl.ds(tid * TPC, TPC)], didx)

        _fill_zero_2d(gbufs[0], CHUNK, d)

        def zero_chunk(k, carry):
            t = s + NS * k
            pltpu.sync_copy(gbufs[0], acc.at[pl.ds(t * CHUNK, CHUNK)])
            return carry

        lax.fori_loop(0, NODE_SLOTS, zero_chunk, 0)
        plsc.subcore_barrier()

        def body(k, carry):
            pltpu.sync_copy(table.at[sidx.at[k]], gbufs[0])
            pltpu.sync_copy(gbufs[0], acc.at[didx.at[k]], add=True)
            return carry

        lax.fori_loop(0, TPC, body, 0)
        plsc.subcore_barrier()

        def wb_chunk(k, carry):
            r0 = (s + NS * k) * CHUNK
            pltpu.sync_copy(acc.at[pl.ds(r0, CHUNK)], gbufs[0])
            pltpu.sync_copy(gbufs[0], out.at[c, pl.ds(r0, CHUNK)])
            return carry

        lax.fori_loop(0, NODE_SLOTS, wb_chunk, 0)

    return agg


_agg_hid = _make_agg(F_HID)


@functools.partial(
    pl.kernel,
    out_type=(
        jax.ShapeDtypeStruct((NC, N_PAD), jnp.float32),
        jax.ShapeDtypeStruct((NC, N_PAD), jnp.float32),
    ),
    mesh=_sc_mesh(),
    scratch_types=[
        pltpu.VMEM((TPC, CHUNK), jnp.int32),
        pltpu.VMEM((TPC, CHUNK), jnp.int32),
        pltpu.VMEM((CHUNK,), jnp.float32),
        pltpu.VMEM((CHUNK,), jnp.float32),
        pltpu.SemaphoreType.DMA,
        pltpu.SemaphoreType.DMA,
        pltpu.VMEM_SHARED((N_PAD,), jnp.float32),
        pltpu.VMEM_SHARED((N_PAD,), jnp.float32),
    ],
)
def _deg(edges, out_o, out_i, sidx, didx, ones, buf, sem_o, sem_i, acc_o, acc_i):
    c = lax.axis_index("c")
    s = lax.axis_index("s")
    tid = c * NS + s
    one = jnp.ones((16,), jnp.float32)
    z = jnp.zeros((16,), jnp.float32)
    for j in range(CHUNK // 16):
        ones[pl.ds(j * 16, 16)] = one
        buf[pl.ds(j * 16, 16)] = z

    pltpu.sync_copy(edges.at[0, pl.ds(tid * TPC, TPC)], sidx)
    pltpu.sync_copy(edges.at[1, pl.ds(tid * TPC, TPC)], didx)

    def zero_chunk(k, carry):
        t = s + NS * k
        pltpu.sync_copy(buf, acc_o.at[pl.ds(t * CHUNK, CHUNK)])
        pltpu.sync_copy(buf, acc_i.at[pl.ds(t * CHUNK, CHUNK)])
        return carry

    lax.fori_loop(0, NODE_SLOTS, zero_chunk, 0)
    plsc.subcore_barrier()

    def fire(k, carry):
        pltpu.make_async_copy(ones, acc_o.at[sidx.at[k]], sem_o).start(add=True)
        pltpu.make_async_copy(ones, acc_i.at[didx.at[k]], sem_i).start(add=True)
        return carry

    lax.fori_loop(0, TPC, fire, 0)

    def drain(k, carry):
        pltpu.make_async_copy(ones, acc_o.at[sidx.at[0]], sem_o).wait()
        pltpu.make_async_copy(ones, acc_i.at[didx.at[0]], sem_i).wait()
        return carry

    lax.fori_loop(0, TPC, drain, 0)
    plsc.subcore_barrier()

    def wb_chunk(k, carry):
        r0 = (s + NS * k) * CHUNK
        pltpu.sync_copy(acc_o.at[pl.ds(r0, CHUNK)], buf)
        pltpu.sync_copy(buf, out_o.at[c, pl.ds(r0, CHUNK)])
        pltpu.sync_copy(acc_i.at[pl.ds(r0, CHUNK)], buf)
        pltpu.sync_copy(buf, out_i.at[c, pl.ds(r0, CHUNK)])
        return carry

    lax.fori_loop(0, NODE_SLOTS, wb_chunk, 0)


def _norm(deg):
    return jnp.where(deg > 0, lax.rsqrt(jnp.maximum(deg, 1.0)), 0.0)


def _tc_first(features, deg_out, W):
    def body(x_ref, d_ref, w_ref, o_ref):
        ns = _norm(d_ref[...])
        o_ref[...] = jnp.dot(
            x_ref[...] * ns, w_ref[...], preferred_element_type=jnp.float32
        )

    return pl.pallas_call(
        body,
        grid=(N_NODES // ROW_BLK,),
        in_specs=[
            pl.BlockSpec((ROW_BLK, F_IN), lambda i: (i, 0)),
            pl.BlockSpec((ROW_BLK, 1), lambda i: (i, 0)),
            pl.BlockSpec((F_IN, F_HID), lambda i: (0, 0)),
        ],
        out_specs=pl.BlockSpec((ROW_BLK, F_HID), lambda i: (i, 0)),
        out_shape=jax.ShapeDtypeStruct((N_NODES, F_HID), jnp.float32),
    )(features, deg_out, W)


def _tc_mid(parts, deg_in, b, deg_out, W, d_out):
    def body(p_ref, di_ref, b_ref, do_ref, w_ref, o_ref):
        nd = _norm(di_ref[...])
        ns = _norm(do_ref[...])
        h = (p_ref[0] + p_ref[1]) * nd + b_ref[...]
        h = jnp.maximum(h, 0.0)
        o_ref[...] = jnp.dot(h * ns, w_ref[...], preferred_element_type=jnp.float32)

    return pl.pallas_call(
        body,
        grid=(N_NODES // ROW_BLK,),
        in_specs=[
            pl.BlockSpec((NC, ROW_BLK, F_HID), lambda i: (0, i, 0)),
            pl.BlockSpec((ROW_BLK, 1), lambda i: (i, 0)),
            pl.BlockSpec((1, F_HID), lambda i: (0, 0)),
            pl.BlockSpec((ROW_BLK, 1), lambda i: (i, 0)),
            pl.BlockSpec((F_HID, d_out), lambda i: (0, 0)),
        ],
        out_specs=pl.BlockSpec((ROW_BLK, d_out), lambda i: (i, 0)),
        out_shape=jax.ShapeDtypeStruct((N_NODES, d_out), jnp.float32),
    )(parts, deg_in, b, deg_out, W)


def _tc_final(parts, deg_in, b):
    def body(p_ref, di_ref, b_ref, o_ref):
        nd = _norm(di_ref[...])
        o_ref[...] = (p_ref[0] + p_ref[1]) * nd + b_ref[...]

    return pl.pallas_call(
        body,
        grid=(N_NODES // ROW_BLK,),
        in_specs=[
            pl.BlockSpec((NC, ROW_BLK, F_OUT_PAD), lambda i: (0, i, 0)),
            pl.BlockSpec((ROW_BLK, 1), lambda i: (i, 0)),
            pl.BlockSpec((1, F_OUT_PAD), lambda i: (0, 0)),
        ],
        out_specs=pl.BlockSpec((ROW_BLK, F_OUT_PAD), lambda i: (i, 0)),
        out_shape=jax.ShapeDtypeStruct((N_NODES, F_OUT_PAD), jnp.float32),
    )(parts, deg_in, b)


def kernel(features, edge_index, W1, b1, W2, b2, W3, b3):
    W3p = jnp.pad(W3, ((0, 0), (0, F_OUT_PAD - F_OUT)))
    b3p = jnp.pad(b3, (0, F_OUT_PAD - F_OUT))

    n_fill = E_PAD - N_EDGES
    fill = jnp.stack(
        [
            jnp.zeros((n_fill,), jnp.int32),
            jnp.full((n_fill,), N_NODES, jnp.int32),
        ]
    )
    edges3 = jnp.concatenate([edge_index, fill], axis=1).reshape(2, E_PAD // CHUNK, CHUNK)

    do_parts, di_parts = _deg(edges3)
    deg_out = (do_parts[0, :N_NODES] + do_parts[1, :N_NODES]).reshape(N_NODES, 1)
    deg_in = (di_parts[0, :N_NODES] + di_parts[1, :N_NODES]).reshape(N_NODES, 1)

    h = _tc_first(features, deg_out, W1)
    parts = _agg_hid(h, edges3)[:, :N_NODES]
    h = _tc_mid(parts, deg_in, b1.reshape(1, -1), deg_out, W2, F_HID)
    parts = _agg_hid(h, edges3)[:, :N_NODES]
    h = _tc_mid(parts, deg_in, b2.reshape(1, -1), deg_out, W3p, F_OUT_PAD)
    parts = _agg_hid(h, edges3)[:, :N_NODES]
    out = _tc_final(parts, deg_in, b3p.reshape(1, -1))
    return out[:, :F_OUT]


# speedup vs baseline: 1.2734x; 1.2734x over previous
import functools

import jax
import jax.numpy as jnp
from jax import lax
from jax.experimental import pallas as pl
from jax.experimental.pallas import tpu as pltpu
from jax.experimental.pallas import tpu_sc as plsc

N_NODES = 10000
N_EDGES = 320000
F_IN = 128
F_HID = 128
F_OUT = 40
F_OUT_PAD = 128

NC = 2
NS = 16
NW = NC * NS
CHUNK = 128
TPC = 80
E_PAD = NW * TPC * CHUNK
NODE_SLOTS = 5
N_PAD = NODE_SLOTS * NS * CHUNK
NB = 4

ROW_BLK = 1000


def _sc_mesh():
    return plsc.VectorSubcoreMesh(
        core_axis_name="c", subcore_axis_name="s", num_cores=NC, num_subcores=NS
    )


def _fill_zero_2d(buf, rows, d):
    z = jnp.zeros((16,), jnp.float32)

    def row(i, carry):
        for j in range(d // 16):
            buf[i, pl.ds(j * 16, 16)] = z
        return carry

    lax.fori_loop(0, rows, row, 0)


def _make_agg(d):
    @functools.partial(
        pl.kernel,
        out_type=jax.ShapeDtypeStruct((NC, N_PAD, d), jnp.float32),
        mesh=_sc_mesh(),
        scratch_types=[
            pltpu.VMEM((2, CHUNK), jnp.int32),
            pltpu.VMEM((CHUNK, d), jnp.float32),
            pltpu.VMEM_SHARED((N_PAD, d), jnp.float32),
        ],
    )
    def agg(table, edges, out, eidx, gbuf, acc):
        c = lax.axis_index("c")
        s = lax.axis_index("s")

        _fill_zero_2d(gbuf, CHUNK, d)

        def zero_chunk(k, carry):
            t = s + NS * k
            pltpu.sync_copy(gbuf, acc.at[pl.ds(t * CHUNK, CHUNK)])
            return carry

        lax.fori_loop(0, NODE_SLOTS, zero_chunk, 0)
        plsc.subcore_barrier()

        def body(k, carry):
            chunk = c * (TPC * NS) + s + k * NS
            pltpu.sync_copy(edges.at[chunk], eidx)
            pltpu.sync_copy(table.at[eidx.at[0]], gbuf)
            pltpu.sync_copy(gbuf, acc.at[eidx.at[1]], add=True)
            return carry

        lax.fori_loop(0, TPC, body, 0)
        plsc.subcore_barrier()

        def wb_chunk(k, carry):
            r0 = (s + NS * k) * CHUNK
            pltpu.sync_copy(acc.at[pl.ds(r0, CHUNK)], gbuf)
            pltpu.sync_copy(gbuf, out.at[c, pl.ds(r0, CHUNK)])
            return carry

        lax.fori_loop(0, NODE_SLOTS, wb_chunk, 0)

    return agg


_agg_hid = _make_agg(F_HID)


@functools.partial(
    pl.kernel,
    out_type=(
        jax.ShapeDtypeStruct((NC, N_PAD), jnp.float32),
        jax.ShapeDtypeStruct((NC, N_PAD), jnp.float32),
    ),
    mesh=_sc_mesh(),
    scratch_types=[
        pltpu.VMEM((TPC, CHUNK), jnp.int32),
        pltpu.VMEM((TPC, CHUNK), jnp.int32),
        pltpu.VMEM((CHUNK,), jnp.float32),
        pltpu.VMEM((CHUNK,), jnp.float32),
        pltpu.SemaphoreType.DMA,
        pltpu.SemaphoreType.DMA,
        pltpu.VMEM_SHARED((N_PAD,), jnp.float32),
        pltpu.VMEM_SHARED((N_PAD,), jnp.float32),
    ],
)
def _deg(edges, out_o, out_i, sidx, didx, ones, buf, sem_o, sem_i, acc_o, acc_i):
    c = lax.axis_index("c")
    s = lax.axis_index("s")
    tid = c * NS + s
    one = jnp.ones((16,), jnp.float32)
    z = jnp.zeros((16,), jnp.float32)
    for j in range(CHUNK // 16):
        ones[pl.ds(j * 16, 16)] = one
        buf[pl.ds(j * 16, 16)] = z

    pltpu.sync_copy(edges.at[pl.ds(tid * TPC, TPC), 0], sidx)
    pltpu.sync_copy(edges.at[pl.ds(tid * TPC, TPC), 1], didx)

    def zero_chunk(k, carry):
        t = s + NS * k
        pltpu.sync_copy(buf, acc_o.at[pl.ds(t * CHUNK, CHUNK)])
        pltpu.sync_copy(buf, acc_i.at[pl.ds(t * CHUNK, CHUNK)])
        return carry

    lax.fori_loop(0, NODE_SLOTS, zero_chunk, 0)
    plsc.subcore_barrier()

    def fire(k, carry):
        pltpu.make_async_copy(ones, acc_o.at[sidx.at[k]], sem_o).start(add=True)
        pltpu.make_async_copy(ones, acc_i.at[didx.at[k]], sem_i).start(add=True)
        return carry

    lax.fori_loop(0, TPC, fire, 0)

    def drain(k, carry):
        pltpu.make_async_copy(ones, acc_o.at[sidx.at[0]], sem_o).wait()
        pltpu.make_async_copy(ones, acc_i.at[didx.at[0]], sem_i).wait()
        return carry

    lax.fori_loop(0, TPC, drain, 0)
    plsc.subcore_barrier()

    def wb_chunk(k, carry):
        r0 = (s + NS * k) * CHUNK
        pltpu.sync_copy(acc_o.at[pl.ds(r0, CHUNK)], buf)
        pltpu.sync_copy(buf, out_o.at[c, pl.ds(r0, CHUNK)])
        pltpu.sync_copy(acc_i.at[pl.ds(r0, CHUNK)], buf)
        pltpu.sync_copy(buf, out_i.at[c, pl.ds(r0, CHUNK)])
        return carry

    lax.fori_loop(0, NODE_SLOTS, wb_chunk, 0)


def _norm(deg):
    return jnp.where(deg > 0, lax.rsqrt(jnp.maximum(deg, 1.0)), 0.0)


def _tc_first(features, deg_out, W):
    def body(x_ref, d_ref, w_ref, o_ref):
        ns = _norm(d_ref[...])
        o_ref[...] = jnp.dot(
            x_ref[...] * ns, w_ref[...], preferred_element_type=jnp.float32
        )

    return pl.pallas_call(
        body,
        grid=(N_NODES // ROW_BLK,),
        in_specs=[
            pl.BlockSpec((ROW_BLK, F_IN), lambda i: (i, 0)),
            pl.BlockSpec((ROW_BLK, 1), lambda i: (i, 0)),
            pl.BlockSpec((F_IN, F_HID), lambda i: (0, 0)),
        ],
        out_specs=pl.BlockSpec((ROW_BLK, F_HID), lambda i: (i, 0)),
        out_shape=jax.ShapeDtypeStruct((N_NODES, F_HID), jnp.float32),
    )(features, deg_out, W)


def _tc_mid(parts, deg_in, b, deg_out, W, d_out):
    def body(p_ref, di_ref, b_ref, do_ref, w_ref, o_ref):
        nd = _norm(di_ref[...])
        ns = _norm(do_ref[...])
        h = (p_ref[0] + p_ref[1]) * nd + b_ref[...]
        h = jnp.maximum(h, 0.0)
        o_ref[...] = jnp.dot(h * ns, w_ref[...], preferred_element_type=jnp.float32)

    return pl.pallas_call(
        body,
        grid=(N_NODES // ROW_BLK,),
        in_specs=[
            pl.BlockSpec((NC, ROW_BLK, F_HID), lambda i: (0, i, 0)),
            pl.BlockSpec((ROW_BLK, 1), lambda i: (i, 0)),
            pl.BlockSpec((1, F_HID), lambda i: (0, 0)),
            pl.BlockSpec((ROW_BLK, 1), lambda i: (i, 0)),
            pl.BlockSpec((F_HID, d_out), lambda i: (0, 0)),
        ],
        out_specs=pl.BlockSpec((ROW_BLK, d_out), lambda i: (i, 0)),
        out_shape=jax.ShapeDtypeStruct((N_NODES, d_out), jnp.float32),
    )(parts, deg_in, b, deg_out, W)


def _tc_final(parts, deg_in, b):
    def body(p_ref, di_ref, b_ref, o_ref):
        nd = _norm(di_ref[...])
        o_ref[...] = (p_ref[0] + p_ref[1]) * nd + b_ref[...]

    return pl.pallas_call(
        body,
        grid=(N_NODES // ROW_BLK,),
        in_specs=[
            pl.BlockSpec((NC, ROW_BLK, F_OUT_PAD), lambda i: (0, i, 0)),
            pl.BlockSpec((ROW_BLK, 1), lambda i: (i, 0)),
            pl.BlockSpec((1, F_OUT_PAD), lambda i: (0, 0)),
        ],
        out_specs=pl.BlockSpec((ROW_BLK, F_OUT_PAD), lambda i: (i, 0)),
        out_shape=jax.ShapeDtypeStruct((N_NODES, F_OUT_PAD), jnp.float32),
    )(parts, deg_in, b)


def kernel(features, edge_index, W1, b1, W2, b2, W3, b3):
    W3p = jnp.pad(W3, ((0, 0), (0, F_OUT_PAD - F_OUT)))
    b3p = jnp.pad(b3, (0, F_OUT_PAD - F_OUT))

    n_fill = E_PAD - N_EDGES
    fill = jnp.stack(
        [
            jnp.zeros((n_fill,), jnp.int32),
            jnp.full((n_fill,), N_NODES, jnp.int32),
        ]
    )
    edges3 = jnp.concatenate([edge_index, fill], axis=1).reshape(2, E_PAD // CHUNK, CHUNK)
    edges4 = jnp.transpose(edges3, (1, 0, 2))

    do_parts, di_parts = _deg(edges4)
    deg_out = (do_parts[0, :N_NODES] + do_parts[1, :N_NODES]).reshape(N_NODES, 1)
    deg_in = (di_parts[0, :N_NODES] + di_parts[1, :N_NODES]).reshape(N_NODES, 1)

    h = _tc_first(features, deg_out, W1)
    parts = _agg_hid(h, edges4)[:, :N_NODES]
    h = _tc_mid(parts, deg_in, b1.reshape(1, -1), deg_out, W2, F_HID)
    parts = _agg_hid(h, edges4)[:, :N_NODES]
    h = _tc_mid(parts, deg_in, b2.reshape(1, -1), deg_out, W3p, F_OUT_PAD)
    parts = _agg_hid(h, edges4)[:, :N_NODES]
    out = _tc_final(parts, deg_in, b3p.reshape(1, -1))
    return out[:, :F_OUT]


# speedup vs baseline: 2.7702x; 2.1755x over previous
import functools

import jax
import jax.numpy as jnp
from jax import lax
from jax.experimental import pallas as pl
from jax.experimental.pallas import tpu as pltpu
from jax.experimental.pallas import tpu_sc as plsc

N_NODES = 10000
N_EDGES = 320000
F_IN = 128
F_HID = 128
F_OUT = 40
F_OUT_PAD = 128

NC = 2
NS = 16
NW = NC * NS
CHUNK = 128
TPC = 80
E_PAD = NW * TPC * CHUNK
NODE_SLOTS = 5
N_PAD = NODE_SLOTS * NS * CHUNK
NB = 4

ROW_BLK = 1024


def _sc_mesh():
    return plsc.VectorSubcoreMesh(
        core_axis_name="c", subcore_axis_name="s", num_cores=NC, num_subcores=NS
    )


def _fill_zero_2d(buf, rows, d):
    z = jnp.zeros((16,), jnp.float32)

    def row(i, carry):
        for j in range(d // 16):
            buf[i, pl.ds(j * 16, 16)] = z
        return carry

    lax.fori_loop(0, rows, row, 0)


def _make_agg(d):
    @functools.partial(
        pl.kernel,
        out_type=jax.ShapeDtypeStruct((NC, N_PAD, d), jnp.float32),
        mesh=_sc_mesh(),
        scratch_types=[
            pltpu.VMEM((2, CHUNK), jnp.int32),
            pltpu.VMEM((CHUNK, d), jnp.float32),
            pltpu.VMEM_SHARED((N_PAD, d), jnp.float32),
        ],
    )
    def agg(table, edges, out, eidx, gbuf, acc):
        c = lax.axis_index("c")
        s = lax.axis_index("s")

        _fill_zero_2d(gbuf, CHUNK, d)

        def zero_chunk(k, carry):
            t = s + NS * k
            pltpu.sync_copy(gbuf, acc.at[pl.ds(t * CHUNK, CHUNK)])
            return carry

        lax.fori_loop(0, NODE_SLOTS, zero_chunk, 0)
        plsc.subcore_barrier()

        def body(k, carry):
            chunk = c * (TPC * NS) + s + k * NS
            pltpu.sync_copy(edges.at[chunk], eidx)
            pltpu.sync_copy(table.at[eidx.at[0]], gbuf)
            pltpu.sync_copy(gbuf, acc.at[eidx.at[1]], add=True)
            return carry

        lax.fori_loop(0, TPC, body, 0)
        plsc.subcore_barrier()

        def wb_chunk(k, carry):
            r0 = (s + NS * k) * CHUNK
            pltpu.sync_copy(acc.at[pl.ds(r0, CHUNK)], gbuf)
            pltpu.sync_copy(gbuf, out.at[c, pl.ds(r0, CHUNK)])
            return carry

        lax.fori_loop(0, NODE_SLOTS, wb_chunk, 0)

    return agg


_agg_hid = _make_agg(F_HID)


@functools.partial(
    pl.kernel,
    out_type=(
        jax.ShapeDtypeStruct((NC, N_PAD), jnp.float32),
        jax.ShapeDtypeStruct((NC, N_PAD), jnp.float32),
    ),
    mesh=_sc_mesh(),
    scratch_types=[
        pltpu.VMEM((TPC, CHUNK), jnp.int32),
        pltpu.VMEM((TPC, CHUNK), jnp.int32),
        pltpu.VMEM((CHUNK,), jnp.float32),
        pltpu.VMEM((CHUNK,), jnp.float32),
        pltpu.SemaphoreType.DMA,
        pltpu.SemaphoreType.DMA,
        pltpu.VMEM_SHARED((N_PAD,), jnp.float32),
        pltpu.VMEM_SHARED((N_PAD,), jnp.float32),
    ],
)
def _deg(edges, out_o, out_i, sidx, didx, ones, buf, sem_o, sem_i, acc_o, acc_i):
    c = lax.axis_index("c")
    s = lax.axis_index("s")
    tid = c * NS + s
    one = jnp.ones((16,), jnp.float32)
    z = jnp.zeros((16,), jnp.float32)
    for j in range(CHUNK // 16):
        ones[pl.ds(j * 16, 16)] = one
        buf[pl.ds(j * 16, 16)] = z

    pltpu.sync_copy(edges.at[pl.ds(tid * TPC, TPC), 0], sidx)
    pltpu.sync_copy(edges.at[pl.ds(tid * TPC, TPC), 1], didx)

    def zero_chunk(k, carry):
        t = s + NS * k
        pltpu.sync_copy(buf, acc_o.at[pl.ds(t * CHUNK, CHUNK)])
        pltpu.sync_copy(buf, acc_i.at[pl.ds(t * CHUNK, CHUNK)])
        return carry

    lax.fori_loop(0, NODE_SLOTS, zero_chunk, 0)
    plsc.subcore_barrier()

    def fire(k, carry):
        pltpu.make_async_copy(ones, acc_o.at[sidx.at[k]], sem_o).start(add=True)
        pltpu.make_async_copy(ones, acc_i.at[didx.at[k]], sem_i).start(add=True)
        return carry

    lax.fori_loop(0, TPC, fire, 0)

    def drain(k, carry):
        pltpu.make_async_copy(ones, acc_o.at[sidx.at[0]], sem_o).wait()
        pltpu.make_async_copy(ones, acc_i.at[didx.at[0]], sem_i).wait()
        return carry

    lax.fori_loop(0, TPC, drain, 0)
    plsc.subcore_barrier()

    def wb_chunk(k, carry):
        r0 = (s + NS * k) * CHUNK
        pltpu.sync_copy(acc_o.at[pl.ds(r0, CHUNK)], buf)
        pltpu.sync_copy(buf, out_o.at[c, pl.ds(r0, CHUNK)])
        pltpu.sync_copy(acc_i.at[pl.ds(r0, CHUNK)], buf)
        pltpu.sync_copy(buf, out_i.at[c, pl.ds(r0, CHUNK)])
        return carry

    lax.fori_loop(0, NODE_SLOTS, wb_chunk, 0)


def _norm(deg):
    return jnp.where(deg > 0, lax.rsqrt(jnp.maximum(deg, 1.0)), 0.0)


def _tc_first(features, deg_out, W):
    def body(x_ref, d_ref, w_ref, o_ref):
        ns = _norm(d_ref[...])
        o_ref[...] = jnp.dot(
            x_ref[...] * ns, w_ref[...], preferred_element_type=jnp.float32
        )

    return pl.pallas_call(
        body,
        grid=(N_PAD // ROW_BLK,),
        in_specs=[
            pl.BlockSpec((ROW_BLK, F_IN), lambda i: (i, 0)),
            pl.BlockSpec((ROW_BLK, 1), lambda i: (i, 0)),
            pl.BlockSpec((F_IN, F_HID), lambda i: (0, 0)),
        ],
        out_specs=pl.BlockSpec((ROW_BLK, F_HID), lambda i: (i, 0)),
        out_shape=jax.ShapeDtypeStruct((N_PAD, F_HID), jnp.float32),
    )(features, deg_out, W)


def _tc_mid(parts, deg_in, b, deg_out, W, d_out):
    def body(p_ref, di_ref, b_ref, do_ref, w_ref, o_ref):
        nd = _norm(di_ref[...])
        ns = _norm(do_ref[...])
        h = (p_ref[0] + p_ref[1]) * nd + b_ref[...]
        h = jnp.maximum(h, 0.0)
        o_ref[...] = jnp.dot(h * ns, w_ref[...], preferred_element_type=jnp.float32)

    return pl.pallas_call(
        body,
        grid=(N_PAD // ROW_BLK,),
        in_specs=[
            pl.BlockSpec((NC, ROW_BLK, F_HID), lambda i: (0, i, 0)),
            pl.BlockSpec((ROW_BLK, 1), lambda i: (i, 0)),
            pl.BlockSpec((1, F_HID), lambda i: (0, 0)),
            pl.BlockSpec((ROW_BLK, 1), lambda i: (i, 0)),
            pl.BlockSpec((F_HID, d_out), lambda i: (0, 0)),
        ],
        out_specs=pl.BlockSpec((ROW_BLK, d_out), lambda i: (i, 0)),
        out_shape=jax.ShapeDtypeStruct((N_PAD, d_out), jnp.float32),
    )(parts, deg_in, b, deg_out, W)


def _tc_final(parts, deg_in, b):
    def body(p_ref, di_ref, b_ref, o_ref):
        nd = _norm(di_ref[...])
        o_ref[...] = (p_ref[0] + p_ref[1]) * nd + b_ref[...]

    return pl.pallas_call(
        body,
        grid=(N_PAD // ROW_BLK,),
        in_specs=[
            pl.BlockSpec((NC, ROW_BLK, F_OUT_PAD), lambda i: (0, i, 0)),
            pl.BlockSpec((ROW_BLK, 1), lambda i: (i, 0)),
            pl.BlockSpec((1, F_OUT_PAD), lambda i: (0, 0)),
        ],
        out_specs=pl.BlockSpec((ROW_BLK, F_OUT_PAD), lambda i: (i, 0)),
        out_shape=jax.ShapeDtypeStruct((N_PAD, F_OUT_PAD), jnp.float32),
    )(parts, deg_in, b)


def kernel(features, edge_index, W1, b1, W2, b2, W3, b3):
    W3p = jnp.pad(W3, ((0, 0), (0, F_OUT_PAD - F_OUT)))
    b3p = jnp.pad(b3, (0, F_OUT_PAD - F_OUT))

    n_fill = E_PAD - N_EDGES
    trash = N_NODES + jnp.arange(n_fill, dtype=jnp.int32) % (N_PAD - N_NODES)
    fill = jnp.stack([trash, trash])
    edges4 = jnp.transpose(
        jnp.concatenate([edge_index, fill], axis=1).reshape(2, E_PAD // CHUNK, CHUNK),
        (1, 0, 2),
    )
    features_p = jnp.pad(features, ((0, N_PAD - N_NODES), (0, 0)))

    do_parts, di_parts = _deg(edges4)
    deg_out = (do_parts[0] + do_parts[1]).reshape(N_PAD, 1)
    deg_in = (di_parts[0] + di_parts[1]).reshape(N_PAD, 1)

    h = _tc_first(features_p, deg_out, W1)
    parts = _agg_hid(h, edges4)
    h = _tc_mid(parts, deg_in, b1.reshape(1, -1), deg_out, W2, F_HID)
    parts = _agg_hid(h, edges4)
    h = _tc_mid(parts, deg_in, b2.reshape(1, -1), deg_out, W3p, F_OUT_PAD)
    parts = _agg_hid(h, edges4)
    out = _tc_final(parts, deg_in, b3p.reshape(1, -1))
    return out[:N_NODES, :F_OUT]


# speedup vs baseline: 3.2334x; 1.1672x over previous
import functools

import jax
import jax.numpy as jnp
from jax import lax
from jax.experimental import pallas as pl
from jax.experimental.pallas import tpu as pltpu
from jax.experimental.pallas import tpu_sc as plsc

N_NODES = 10000
N_EDGES = 320000
F_IN = 128
F_HID = 128
F_OUT = 40
F_OUT_PAD = 128

NC = 2
NS = 16
NW = NC * NS
CHUNK = 128
TPC = 80
E_PAD = NW * TPC * CHUNK
NODE_SLOTS = 5
N_PAD = NODE_SLOTS * NS * CHUNK
NB = 4

ROW_BLK = 1024


def _sc_mesh():
    return plsc.VectorSubcoreMesh(
        core_axis_name="c", subcore_axis_name="s", num_cores=NC, num_subcores=NS
    )


def _fill_zero_2d(buf, rows, d):
    z = jnp.zeros((16,), jnp.float32)

    def row(i, carry):
        for j in range(d // 16):
            buf[i, pl.ds(j * 16, 16)] = z
        return carry

    lax.fori_loop(0, rows, row, 0)


def _make_agg(d):
    @functools.partial(
        pl.kernel,
        out_type=jax.ShapeDtypeStruct((NC, N_PAD, d), jnp.float32),
        mesh=_sc_mesh(),
        scratch_types=[
            pltpu.VMEM((TPC, CHUNK), jnp.int32),
            pltpu.VMEM((TPC, CHUNK), jnp.int32),
            pltpu.VMEM((CHUNK, d), jnp.float32),
            pltpu.VMEM_SHARED((N_PAD, d), jnp.float32),
        ],
    )
    def agg(table, edges, out, sidx, didx, gbuf, acc):
        c = lax.axis_index("c")
        s = lax.axis_index("s")
        tid = c * NS + s
        pltpu.sync_copy(edges.at[pl.ds(tid * TPC, TPC), 0], sidx)
        pltpu.sync_copy(edges.at[pl.ds(tid * TPC, TPC), 1], didx)

        _fill_zero_2d(gbuf, CHUNK, d)

        def zero_chunk(k, carry):
            t = s + NS * k
            pltpu.sync_copy(gbuf, acc.at[pl.ds(t * CHUNK, CHUNK)])
            return carry

        lax.fori_loop(0, NODE_SLOTS, zero_chunk, 0)
        plsc.subcore_barrier()

        def body(k, carry):
            pltpu.sync_copy(table.at[sidx.at[k]], gbuf)
            pltpu.sync_copy(gbuf, acc.at[didx.at[k]], add=True)
            return carry

        lax.fori_loop(0, TPC, body, 0)
        plsc.subcore_barrier()

        def wb_chunk(k, carry):
            r0 = (s + NS * k) * CHUNK
            pltpu.sync_copy(acc.at[pl.ds(r0, CHUNK)], gbuf)
            pltpu.sync_copy(gbuf, out.at[c, pl.ds(r0, CHUNK)])
            return carry

        lax.fori_loop(0, NODE_SLOTS, wb_chunk, 0)

    return agg


_agg_hid = _make_agg(F_HID)


@functools.partial(
    pl.kernel,
    out_type=(
        jax.ShapeDtypeStruct((NC, N_PAD), jnp.float32),
        jax.ShapeDtypeStruct((NC, N_PAD), jnp.float32),
    ),
    mesh=_sc_mesh(),
    scratch_types=[
        pltpu.VMEM((TPC, CHUNK), jnp.int32),
        pltpu.VMEM((TPC, CHUNK), jnp.int32),
        pltpu.VMEM((CHUNK,), jnp.float32),
        pltpu.VMEM((CHUNK,), jnp.float32),
        pltpu.SemaphoreType.DMA,
        pltpu.SemaphoreType.DMA,
        pltpu.VMEM_SHARED((N_PAD,), jnp.float32),
        pltpu.VMEM_SHARED((N_PAD,), jnp.float32),
    ],
)
def _deg(edges, out_o, out_i, sidx, didx, ones, buf, sem_o, sem_i, acc_o, acc_i):
    c = lax.axis_index("c")
    s = lax.axis_index("s")
    tid = c * NS + s
    one = jnp.ones((16,), jnp.float32)
    z = jnp.zeros((16,), jnp.float32)
    for j in range(CHUNK // 16):
        ones[pl.ds(j * 16, 16)] = one
        buf[pl.ds(j * 16, 16)] = z

    pltpu.sync_copy(edges.at[pl.ds(tid * TPC, TPC), 0], sidx)
    pltpu.sync_copy(edges.at[pl.ds(tid * TPC, TPC), 1], didx)

    def zero_chunk(k, carry):
        t = s + NS * k
        pltpu.sync_copy(buf, acc_o.at[pl.ds(t * CHUNK, CHUNK)])
        pltpu.sync_copy(buf, acc_i.at[pl.ds(t * CHUNK, CHUNK)])
        return carry

    lax.fori_loop(0, NODE_SLOTS, zero_chunk, 0)
    plsc.subcore_barrier()

    def fire(k, carry):
        pltpu.make_async_copy(ones, acc_o.at[sidx.at[k]], sem_o).start(add=True)
        pltpu.make_async_copy(ones, acc_i.at[didx.at[k]], sem_i).start(add=True)
        return carry

    lax.fori_loop(0, TPC, fire, 0)

    def drain(k, carry):
        pltpu.make_async_copy(ones, acc_o.at[sidx.at[0]], sem_o).wait()
        pltpu.make_async_copy(ones, acc_i.at[didx.at[0]], sem_i).wait()
        return carry

    lax.fori_loop(0, TPC, drain, 0)
    plsc.subcore_barrier()

    def wb_chunk(k, carry):
        r0 = (s + NS * k) * CHUNK
        pltpu.sync_copy(acc_o.at[pl.ds(r0, CHUNK)], buf)
        pltpu.sync_copy(buf, out_o.at[c, pl.ds(r0, CHUNK)])
        pltpu.sync_copy(acc_i.at[pl.ds(r0, CHUNK)], buf)
        pltpu.sync_copy(buf, out_i.at[c, pl.ds(r0, CHUNK)])
        return carry

    lax.fori_loop(0, NODE_SLOTS, wb_chunk, 0)


def _norm(deg):
    return jnp.where(deg > 0, lax.rsqrt(jnp.maximum(deg, 1.0)), 0.0)


def _tc_first(features, deg_out, W):
    def body(x_ref, d_ref, w_ref, o_ref):
        ns = _norm(d_ref[...])
        o_ref[...] = jnp.dot(
            x_ref[...] * ns, w_ref[...], preferred_element_type=jnp.float32
        )

    return pl.pallas_call(
        body,
        grid=(N_PAD // ROW_BLK,),
        in_specs=[
            pl.BlockSpec((ROW_BLK, F_IN), lambda i: (i, 0)),
            pl.BlockSpec((ROW_BLK, 1), lambda i: (i, 0)),
            pl.BlockSpec((F_IN, F_HID), lambda i: (0, 0)),
        ],
        out_specs=pl.BlockSpec((ROW_BLK, F_HID), lambda i: (i, 0)),
        out_shape=jax.ShapeDtypeStruct((N_PAD, F_HID), jnp.float32),
    )(features, deg_out, W)


def _tc_mid(parts, deg_in, b, deg_out, W, d_out):
    def body(p_ref, di_ref, b_ref, do_ref, w_ref, o_ref):
        nd = _norm(di_ref[...])
        ns = _norm(do_ref[...])
        h = (p_ref[0] + p_ref[1]) * nd + b_ref[...]
        h = jnp.maximum(h, 0.0)
        o_ref[...] = jnp.dot(h * ns, w_ref[...], preferred_element_type=jnp.float32)

    return pl.pallas_call(
        body,
        grid=(N_PAD // ROW_BLK,),
        in_specs=[
            pl.BlockSpec((NC, ROW_BLK, F_HID), lambda i: (0, i, 0)),
            pl.BlockSpec((ROW_BLK, 1), lambda i: (i, 0)),
            pl.BlockSpec((1, F_HID), lambda i: (0, 0)),
            pl.BlockSpec((ROW_BLK, 1), lambda i: (i, 0)),
            pl.BlockSpec((F_HID, d_out), lambda i: (0, 0)),
        ],
        out_specs=pl.BlockSpec((ROW_BLK, d_out), lambda i: (i, 0)),
        out_shape=jax.ShapeDtypeStruct((N_PAD, d_out), jnp.float32),
    )(parts, deg_in, b, deg_out, W)


def _tc_final(parts, deg_in, b):
    def body(p_ref, di_ref, b_ref, o_ref):
        nd = _norm(di_ref[...])
        o_ref[...] = (p_ref[0] + p_ref[1]) * nd + b_ref[...]

    return pl.pallas_call(
        body,
        grid=(N_PAD // ROW_BLK,),
        in_specs=[
            pl.BlockSpec((NC, ROW_BLK, F_OUT_PAD), lambda i: (0, i, 0)),
            pl.BlockSpec((ROW_BLK, 1), lambda i: (i, 0)),
            pl.BlockSpec((1, F_OUT_PAD), lambda i: (0, 0)),
        ],
        out_specs=pl.BlockSpec((ROW_BLK, F_OUT_PAD), lambda i: (i, 0)),
        out_shape=jax.ShapeDtypeStruct((N_PAD, F_OUT_PAD), jnp.float32),
    )(parts, deg_in, b)


def kernel(features, edge_index, W1, b1, W2, b2, W3, b3):
    W3p = jnp.pad(W3, ((0, 0), (0, F_OUT_PAD - F_OUT)))
    b3p = jnp.pad(b3, (0, F_OUT_PAD - F_OUT))

    n_fill = E_PAD - N_EDGES
    trash = N_NODES + jnp.arange(n_fill, dtype=jnp.int32) % (N_PAD - N_NODES)
    fill = jnp.stack([trash, trash])
    edges4 = jnp.transpose(
        jnp.concatenate([edge_index, fill], axis=1).reshape(2, E_PAD // CHUNK, CHUNK),
        (1, 0, 2),
    )
    features_p = jnp.pad(features, ((0, N_PAD - N_NODES), (0, 0)))

    do_parts, di_parts = _deg(edges4)
    deg_out = (do_parts[0] + do_parts[1]).reshape(N_PAD, 1)
    deg_in = (di_parts[0] + di_parts[1]).reshape(N_PAD, 1)

    h = _tc_first(features_p, deg_out, W1)
    parts = _agg_hid(h, edges4)
    h = _tc_mid(parts, deg_in, b1.reshape(1, -1), deg_out, W2, F_HID)
    parts = _agg_hid(h, edges4)
    h = _tc_mid(parts, deg_in, b2.reshape(1, -1), deg_out, W3p, F_OUT_PAD)
    parts = _agg_hid(h, edges4)
    out = _tc_final(parts, deg_in, b3p.reshape(1, -1))
    return out[:N_NODES, :F_OUT]


# speedup vs baseline: 3.5193x; 1.0884x over previous
import functools

import jax
import jax.numpy as jnp
from jax import lax
from jax.experimental import pallas as pl
from jax.experimental.pallas import tpu as pltpu
from jax.experimental.pallas import tpu_sc as plsc

N_NODES = 10000
N_EDGES = 320000
F_IN = 128
F_HID = 128
F_OUT = 40
F_OUT_PAD = 128
F_NAR = 48

NC = 2
NS = 16
NW = NC * NS
CHUNK = 128
TPC = 80
E_PAD = NW * TPC * CHUNK
NODE_SLOTS = 5
N_PAD = NODE_SLOTS * NS * CHUNK
NB = 4

ROW_BLK = 1024


def _sc_mesh():
    return plsc.VectorSubcoreMesh(
        core_axis_name="c", subcore_axis_name="s", num_cores=NC, num_subcores=NS
    )


def _fill_zero_2d(buf, rows, d):
    z = jnp.zeros((16,), jnp.float32)

    def row(i, carry):
        for j in range(d // 16):
            buf[i, pl.ds(j * 16, 16)] = z
        return carry

    lax.fori_loop(0, rows, row, 0)


def _make_agg(d, tc_tiling=True):
    @functools.partial(
        pl.kernel,
        out_type=jax.ShapeDtypeStruct((NC, N_PAD, d), jnp.float32),
        mesh=_sc_mesh(),
        compiler_params=pltpu.CompilerParams(use_tc_tiling_on_sc=tc_tiling),
        scratch_types=[
            pltpu.VMEM((TPC, CHUNK), jnp.int32),
            pltpu.VMEM((TPC, CHUNK), jnp.int32),
            pltpu.VMEM((CHUNK, d), jnp.float32),
            pltpu.VMEM_SHARED((N_PAD, d), jnp.float32),
        ],
    )
    def agg(table, src_e, dst_e, out, sidx, didx, gbuf, acc):
        c = lax.axis_index("c")
        s = lax.axis_index("s")
        tid = c * NS + s
        pltpu.sync_copy(src_e.at[pl.ds(tid * TPC, TPC)], sidx)
        pltpu.sync_copy(dst_e.at[pl.ds(tid * TPC, TPC)], didx)

        _fill_zero_2d(gbuf, CHUNK, d)

        def zero_chunk(k, carry):
            t = s + NS * k
            pltpu.sync_copy(gbuf, acc.at[pl.ds(t * CHUNK, CHUNK)])
            return carry

        lax.fori_loop(0, NODE_SLOTS, zero_chunk, 0)
        plsc.subcore_barrier()

        def body(k, carry):
            pltpu.sync_copy(table.at[sidx.at[k]], gbuf)
            pltpu.sync_copy(gbuf, acc.at[didx.at[k]], add=True)
            return carry

        lax.fori_loop(0, TPC, body, 0)
        plsc.subcore_barrier()

        def wb_chunk(k, carry):
            r0 = (s + NS * k) * CHUNK
            pltpu.sync_copy(acc.at[pl.ds(r0, CHUNK)], gbuf)
            pltpu.sync_copy(gbuf, out.at[c, pl.ds(r0, CHUNK)])
            return carry

        lax.fori_loop(0, NODE_SLOTS, wb_chunk, 0)

    return agg


_agg_hid = _make_agg(F_HID)
_agg_nar = _make_agg(F_NAR, tc_tiling=False)


@functools.partial(
    pl.kernel,
    out_type=(
        jax.ShapeDtypeStruct((NC, N_PAD), jnp.float32),
        jax.ShapeDtypeStruct((NC, N_PAD), jnp.float32),
    ),
    mesh=_sc_mesh(),
    scratch_types=[
        pltpu.VMEM((TPC, CHUNK), jnp.int32),
        pltpu.VMEM((TPC, CHUNK), jnp.int32),
        pltpu.VMEM((CHUNK,), jnp.float32),
        pltpu.VMEM((CHUNK,), jnp.float32),
        pltpu.SemaphoreType.DMA,
        pltpu.SemaphoreType.DMA,
        pltpu.VMEM_SHARED((N_PAD,), jnp.float32),
        pltpu.VMEM_SHARED((N_PAD,), jnp.float32),
    ],
)
def _deg(src_e, dst_e, out_o, out_i, sidx, didx, ones, buf, sem_o, sem_i, acc_o, acc_i):
    c = lax.axis_index("c")
    s = lax.axis_index("s")
    tid = c * NS + s
    one = jnp.ones((16,), jnp.float32)
    z = jnp.zeros((16,), jnp.float32)
    for j in range(CHUNK // 16):
        ones[pl.ds(j * 16, 16)] = one
        buf[pl.ds(j * 16, 16)] = z

    pltpu.sync_copy(src_e.at[pl.ds(tid * TPC, TPC)], sidx)
    pltpu.sync_copy(dst_e.at[pl.ds(tid * TPC, TPC)], didx)

    def zero_chunk(k, carry):
        t = s + NS * k
        pltpu.sync_copy(buf, acc_o.at[pl.ds(t * CHUNK, CHUNK)])
        pltpu.sync_copy(buf, acc_i.at[pl.ds(t * CHUNK, CHUNK)])
        return carry

    lax.fori_loop(0, NODE_SLOTS, zero_chunk, 0)
    plsc.subcore_barrier()

    def fire(k, carry):
        pltpu.make_async_copy(ones, acc_o.at[sidx.at[k]], sem_o).start(add=True)
        pltpu.make_async_copy(ones, acc_i.at[didx.at[k]], sem_i).start(add=True)
        return carry

    lax.fori_loop(0, TPC, fire, 0)

    def drain(k, carry):
        pltpu.make_async_copy(ones, acc_o.at[sidx.at[0]], sem_o).wait()
        pltpu.make_async_copy(ones, acc_i.at[didx.at[0]], sem_i).wait()
        return carry

    lax.fori_loop(0, TPC, drain, 0)
    plsc.subcore_barrier()

    def wb_chunk(k, carry):
        r0 = (s + NS * k) * CHUNK
        pltpu.sync_copy(acc_o.at[pl.ds(r0, CHUNK)], buf)
        pltpu.sync_copy(buf, out_o.at[c, pl.ds(r0, CHUNK)])
        pltpu.sync_copy(acc_i.at[pl.ds(r0, CHUNK)], buf)
        pltpu.sync_copy(buf, out_i.at[c, pl.ds(r0, CHUNK)])
        return carry

    lax.fori_loop(0, NODE_SLOTS, wb_chunk, 0)


def _norm(deg):
    return jnp.where(deg > 0, lax.rsqrt(jnp.maximum(deg, 1.0)), 0.0)


def _tc_first(features, deg_out, W):
    def body(x_ref, d_ref, w_ref, o_ref):
        ns = _norm(d_ref[...])
        o_ref[...] = jnp.dot(
            x_ref[...] * ns, w_ref[...], preferred_element_type=jnp.float32
        )

    return pl.pallas_call(
        body,
        grid=(N_PAD // ROW_BLK,),
        in_specs=[
            pl.BlockSpec((ROW_BLK, F_IN), lambda i: (i, 0)),
            pl.BlockSpec((ROW_BLK, 1), lambda i: (i, 0)),
            pl.BlockSpec((F_IN, F_HID), lambda i: (0, 0)),
        ],
        out_specs=pl.BlockSpec((ROW_BLK, F_HID), lambda i: (i, 0)),
        out_shape=jax.ShapeDtypeStruct((N_PAD, F_HID), jnp.float32),
    )(features, deg_out, W)


def _tc_mid(parts, deg_in, b, deg_out, W, d_out):
    def body(p_ref, di_ref, b_ref, do_ref, w_ref, o_ref):
        nd = _norm(di_ref[...])
        ns = _norm(do_ref[...])
        h = (p_ref[0] + p_ref[1]) * nd + b_ref[...]
        h = jnp.maximum(h, 0.0)
        o_ref[...] = jnp.dot(h * ns, w_ref[...], preferred_element_type=jnp.float32)

    return pl.pallas_call(
        body,
        grid=(N_PAD // ROW_BLK,),
        in_specs=[
            pl.BlockSpec((NC, ROW_BLK, F_HID), lambda i: (0, i, 0)),
            pl.BlockSpec((ROW_BLK, 1), lambda i: (i, 0)),
            pl.BlockSpec((1, F_HID), lambda i: (0, 0)),
            pl.BlockSpec((ROW_BLK, 1), lambda i: (i, 0)),
            pl.BlockSpec((F_HID, d_out), lambda i: (0, 0)),
        ],
        out_specs=pl.BlockSpec((ROW_BLK, d_out), lambda i: (i, 0)),
        out_shape=jax.ShapeDtypeStruct((N_PAD, d_out), jnp.float32),
    )(parts, deg_in, b, deg_out, W)


def _tc_final(parts, deg_in, b):
    def body(p_ref, di_ref, b_ref, o_ref):
        nd = _norm(di_ref[...])
        o_ref[...] = (p_ref[0] + p_ref[1]) * nd + b_ref[...]

    return pl.pallas_call(
        body,
        grid=(N_PAD // ROW_BLK,),
        in_specs=[
            pl.BlockSpec((NC, ROW_BLK, F_NAR), lambda i: (0, i, 0)),
            pl.BlockSpec((ROW_BLK, 1), lambda i: (i, 0)),
            pl.BlockSpec((1, F_NAR), lambda i: (0, 0)),
        ],
        out_specs=pl.BlockSpec((ROW_BLK, F_NAR), lambda i: (i, 0)),
        out_shape=jax.ShapeDtypeStruct((N_PAD, F_NAR), jnp.float32),
    )(parts, deg_in, b)


def kernel(features, edge_index, W1, b1, W2, b2, W3, b3):
    W3p = jnp.pad(W3, ((0, 0), (0, F_NAR - F_OUT)))
    b3p = jnp.pad(b3, (0, F_NAR - F_OUT))

    n_fill = E_PAD - N_EDGES
    trash = N_NODES + jnp.arange(n_fill, dtype=jnp.int32) % (N_PAD - N_NODES)
    src_e = jnp.concatenate([edge_index[0], trash]).reshape(E_PAD // CHUNK, CHUNK)
    dst_e = jnp.concatenate([edge_index[1], trash]).reshape(E_PAD // CHUNK, CHUNK)
    features_p = jnp.pad(features, ((0, N_PAD - N_NODES), (0, 0)))

    do_parts, di_parts = _deg(src_e, dst_e)
    deg_out = (do_parts[0] + do_parts[1]).reshape(N_PAD, 1)
    deg_in = (di_parts[0] + di_parts[1]).reshape(N_PAD, 1)

    h = _tc_first(features_p, deg_out, W1)
    parts = _agg_hid(h, src_e, dst_e)
    h = _tc_mid(parts, deg_in, b1.reshape(1, -1), deg_out, W2, F_HID)
    parts = _agg_hid(h, src_e, dst_e)
    h = _tc_mid(parts, deg_in, b2.reshape(1, -1), deg_out, W3p, F_NAR)
    parts = _agg_nar(h, src_e, dst_e)
    out = _tc_final(parts, deg_in, b3p.reshape(1, -1))
    return out[:N_NODES, :F_OUT]


# speedup vs baseline: 4.0180x; 1.1417x over previous
import functools

import jax
import jax.numpy as jnp
from jax import lax
from jax.experimental import pallas as pl
from jax.experimental.pallas import tpu as pltpu
from jax.experimental.pallas import tpu_sc as plsc

N_NODES = 10000
N_EDGES = 320000
F_IN = 128
F_HID = 128
F_HALF = 64
F_OUT = 40
F_NAR = 48

NC = 2
NS = 16
NW = NC * NS
CHUNK = 128
TPC = 80
NTPC = 160
E_PAD = NW * TPC * CHUNK
NCHUNK = 128
NODE_SLOTS = 5
N_PAD = NODE_SLOTS * NS * NCHUNK
NB = 4

ROW_BLK = 1024


def _sc_mesh():
    return plsc.VectorSubcoreMesh(
        core_axis_name="c", subcore_axis_name="s", num_cores=NC, num_subcores=NS
    )


def _fill_zero_2d(buf, rows, d):
    z = jnp.zeros((16,), jnp.float32)

    def row(i, carry):
        for j in range(d // 16):
            buf[i, pl.ds(j * 16, 16)] = z
        return carry

    lax.fori_loop(0, rows, row, 0)


@functools.partial(
    pl.kernel,
    out_type=jax.ShapeDtypeStruct((NC, N_PAD, F_HALF), jnp.float32),
    mesh=_sc_mesh(),
    compiler_params=pltpu.CompilerParams(use_tc_tiling_on_sc=False),
    scratch_types=(
        [pltpu.VMEM((NTPC, CHUNK), jnp.int32)] * 2
        + [pltpu.VMEM((CHUNK, F_HALF), jnp.float32)] * NB
        + [pltpu.SemaphoreType.DMA] * (2 * NB)
        + [pltpu.VMEM_SHARED((N_PAD, F_HALF), jnp.float32)]
    ),
)
def _agg_split(table, src_off, dst_e, out, sidx, didx,
               g0, g1, g2, g3, gs0, gs1, gs2, gs3, ss0, ss1, ss2, ss3, acc):
    gbufs = (g0, g1, g2, g3)
    gsems = (gs0, gs1, gs2, gs3)
    ssems = (ss0, ss1, ss2, ss3)
    c = lax.axis_index("c")
    s = lax.axis_index("s")

    pltpu.sync_copy(src_off.at[c, pl.ds(s * NTPC, NTPC)], sidx)
    pltpu.sync_copy(dst_e.at[pl.ds(s * NTPC, NTPC)], didx)

    _fill_zero_2d(gbufs[0], NCHUNK, F_HALF)

    def zero_chunk(k, carry):
        t = s + NS * k
        pltpu.sync_copy(gbufs[0], acc.at[pl.ds(t * NCHUNK, NCHUNK)])
        return carry

    lax.fori_loop(0, NODE_SLOTS, zero_chunk, 0)
    plsc.subcore_barrier()

    def gat(k, b):
        return pltpu.make_async_copy(table.at[sidx.at[k]], gbufs[b], gsems[b])

    def sca(k, b):
        return pltpu.make_async_copy(gbufs[b], acc.at[didx.at[k]], ssems[b])

    for b in range(NB):
        gat(b, b).start()

    def group(g, carry):
        k0 = g * NB
        for b in range(NB):
            gat(k0 + b, b).wait()
            sca(k0 + b, b).start(add=True)
        for b in range(NB):
            sca(k0 + b, b).wait()

            @pl.when(k0 + NB + b < NTPC)
            def _():
                gat(k0 + NB + b, b).start()

        return carry

    lax.fori_loop(0, NTPC // NB, group, 0)
    plsc.subcore_barrier()

    def wb_chunk(k, carry):
        r0 = (s + NS * k) * NCHUNK
        pltpu.sync_copy(acc.at[pl.ds(r0, NCHUNK)], gbufs[0])
        pltpu.sync_copy(gbufs[0], out.at[c, pl.ds(r0, NCHUNK)])
        return carry

    lax.fori_loop(0, NODE_SLOTS, wb_chunk, 0)


@functools.partial(
    pl.kernel,
    out_type=jax.ShapeDtypeStruct((NC, N_PAD, F_NAR), jnp.float32),
    mesh=_sc_mesh(),
    compiler_params=pltpu.CompilerParams(use_tc_tiling_on_sc=False),
    scratch_types=[
        pltpu.VMEM((TPC, CHUNK), jnp.int32),
        pltpu.VMEM((TPC, CHUNK), jnp.int32),
        pltpu.VMEM((CHUNK, F_NAR), jnp.float32),
        pltpu.VMEM_SHARED((N_PAD, F_NAR), jnp.float32),
    ],
)
def _agg_nar(table, src_e, dst_e, out, sidx, didx, gbuf, acc):
    c = lax.axis_index("c")
    s = lax.axis_index("s")
    tid = c * NS + s
    pltpu.sync_copy(src_e.at[pl.ds(tid * TPC, TPC)], sidx)
    pltpu.sync_copy(dst_e.at[pl.ds(tid * TPC, TPC)], didx)

    _fill_zero_2d(gbuf, NCHUNK, F_NAR)

    def zero_chunk(k, carry):
        t = s + NS * k
        pltpu.sync_copy(gbuf, acc.at[pl.ds(t * NCHUNK, NCHUNK)])
        return carry

    lax.fori_loop(0, NODE_SLOTS, zero_chunk, 0)
    plsc.subcore_barrier()

    def body(k, carry):
        pltpu.sync_copy(table.at[sidx.at[k]], gbuf)
        pltpu.sync_copy(gbuf, acc.at[didx.at[k]], add=True)
        return carry

    lax.fori_loop(0, TPC, body, 0)
    plsc.subcore_barrier()

    def wb_chunk(k, carry):
        r0 = (s + NS * k) * NCHUNK
        pltpu.sync_copy(acc.at[pl.ds(r0, NCHUNK)], gbuf)
        pltpu.sync_copy(gbuf, out.at[c, pl.ds(r0, NCHUNK)])
        return carry

    lax.fori_loop(0, NODE_SLOTS, wb_chunk, 0)


@functools.partial(
    pl.kernel,
    out_type=(
        jax.ShapeDtypeStruct((NC, N_PAD), jnp.float32),
        jax.ShapeDtypeStruct((NC, N_PAD), jnp.float32),
    ),
    mesh=_sc_mesh(),
    scratch_types=[
        pltpu.VMEM((TPC, CHUNK), jnp.int32),
        pltpu.VMEM((TPC, CHUNK), jnp.int32),
        pltpu.VMEM((CHUNK,), jnp.float32),
        pltpu.VMEM((CHUNK,), jnp.float32),
        pltpu.SemaphoreType.DMA,
        pltpu.SemaphoreType.DMA,
        pltpu.VMEM_SHARED((N_PAD,), jnp.float32),
        pltpu.VMEM_SHARED((N_PAD,), jnp.float32),
    ],
)
def _deg(src_e, dst_e, out_o, out_i, sidx, didx, ones, buf, sem_o, sem_i, acc_o, acc_i):
    c = lax.axis_index("c")
    s = lax.axis_index("s")
    tid = c * NS + s
    one = jnp.ones((16,), jnp.float32)
    z = jnp.zeros((16,), jnp.float32)
    for j in range(CHUNK // 16):
        ones[pl.ds(j * 16, 16)] = one
        buf[pl.ds(j * 16, 16)] = z

    pltpu.sync_copy(src_e.at[pl.ds(tid * TPC, TPC)], sidx)
    pltpu.sync_copy(dst_e.at[pl.ds(tid * TPC, TPC)], didx)

    def zero_chunk(k, carry):
        t = s + NS * k
        pltpu.sync_copy(buf, acc_o.at[pl.ds(t * NCHUNK, NCHUNK)])
        pltpu.sync_copy(buf, acc_i.at[pl.ds(t * NCHUNK, NCHUNK)])
        return carry

    lax.fori_loop(0, NODE_SLOTS, zero_chunk, 0)
    plsc.subcore_barrier()

    def fire(k, carry):
        pltpu.make_async_copy(ones, acc_o.at[sidx.at[k]], sem_o).start(add=True)
        pltpu.make_async_copy(ones, acc_i.at[didx.at[k]], sem_i).start(add=True)
        return carry

    lax.fori_loop(0, TPC, fire, 0)

    def drain(k, carry):
        pltpu.make_async_copy(ones, acc_o.at[sidx.at[0]], sem_o).wait()
        pltpu.make_async_copy(ones, acc_i.at[didx.at[0]], sem_i).wait()
        return carry

    lax.fori_loop(0, TPC, drain, 0)
    plsc.subcore_barrier()

    def wb_chunk(k, carry):
        r0 = (s + NS * k) * NCHUNK
        pltpu.sync_copy(acc_o.at[pl.ds(r0, NCHUNK)], buf)
        pltpu.sync_copy(buf, out_o.at[c, pl.ds(r0, NCHUNK)])
        pltpu.sync_copy(acc_i.at[pl.ds(r0, NCHUNK)], buf)
        pltpu.sync_copy(buf, out_i.at[c, pl.ds(r0, NCHUNK)])
        return carry

    lax.fori_loop(0, NODE_SLOTS, wb_chunk, 0)


def _norm(deg):
    return jnp.where(deg > 0, lax.rsqrt(jnp.maximum(deg, 1.0)), 0.0)


def _tc_first(features, deg_out, W):
    def body(x_ref, d_ref, w_ref, lo_ref, hi_ref):
        ns = _norm(d_ref[...])
        h = jnp.dot(x_ref[...] * ns, w_ref[...], preferred_element_type=jnp.float32)
        lo_ref[...] = h[:, :F_HALF]
        hi_ref[...] = h[:, F_HALF:]

    return pl.pallas_call(
        body,
        grid=(N_PAD // ROW_BLK,),
        in_specs=[
            pl.BlockSpec((ROW_BLK, F_IN), lambda i: (i, 0)),
            pl.BlockSpec((ROW_BLK, 1), lambda i: (i, 0)),
            pl.BlockSpec((F_IN, F_HID), lambda i: (0, 0)),
        ],
        out_specs=[pl.BlockSpec((ROW_BLK, F_HALF), lambda i: (i, 0))] * 2,
        out_shape=[jax.ShapeDtypeStruct((N_PAD, F_HALF), jnp.float32)] * 2,
    )(features, deg_out, W)


def _tc_mid(parts, deg_in, b, deg_out, W, d_out, split_out):
    def body(p_ref, di_ref, b_ref, do_ref, w_ref, *o_refs):
        nd = _norm(di_ref[...])
        ns = _norm(do_ref[...])
        p = jnp.concatenate([p_ref[0], p_ref[1]], axis=1)
        h = jnp.maximum(p * nd + b_ref[...], 0.0)
        r = jnp.dot(h * ns, w_ref[...], preferred_element_type=jnp.float32)
        if split_out:
            o_refs[0][...] = r[:, :F_HALF]
            o_refs[1][...] = r[:, F_HALF:]
        else:
            o_refs[0][...] = r

    if split_out:
        out_specs = [pl.BlockSpec((ROW_BLK, F_HALF), lambda i: (i, 0))] * 2
        out_shape = [jax.ShapeDtypeStruct((N_PAD, F_HALF), jnp.float32)] * 2
    else:
        out_specs = pl.BlockSpec((ROW_BLK, d_out), lambda i: (i, 0))
        out_shape = jax.ShapeDtypeStruct((N_PAD, d_out), jnp.float32)

    return pl.pallas_call(
        body,
        grid=(N_PAD // ROW_BLK,),
        in_specs=[
            pl.BlockSpec((NC, ROW_BLK, F_HALF), lambda i: (0, i, 0)),
            pl.BlockSpec((ROW_BLK, 1), lambda i: (i, 0)),
            pl.BlockSpec((1, F_HID), lambda i: (0, 0)),
            pl.BlockSpec((ROW_BLK, 1), lambda i: (i, 0)),
            pl.BlockSpec((F_HID, d_out), lambda i: (0, 0)),
        ],
        out_specs=out_specs,
        out_shape=out_shape,
    )(parts, deg_in, b, deg_out, W)


def _tc_final(parts, deg_in, b):
    def body(p_ref, di_ref, b_ref, o_ref):
        nd = _norm(di_ref[...])
        o_ref[...] = (p_ref[0] + p_ref[1]) * nd + b_ref[...]

    return pl.pallas_call(
        body,
        grid=(N_PAD // ROW_BLK,),
        in_specs=[
            pl.BlockSpec((NC, ROW_BLK, F_NAR), lambda i: (0, i, 0)),
            pl.BlockSpec((ROW_BLK, 1), lambda i: (i, 0)),
            pl.BlockSpec((1, F_NAR), lambda i: (0, 0)),
        ],
        out_specs=pl.BlockSpec((ROW_BLK, F_NAR), lambda i: (i, 0)),
        out_shape=jax.ShapeDtypeStruct((N_PAD, F_NAR), jnp.float32),
    )(parts, deg_in, b)


def kernel(features, edge_index, W1, b1, W2, b2, W3, b3):
    W3p = jnp.pad(W3, ((0, 0), (0, F_NAR - F_OUT)))
    b3p = jnp.pad(b3, (0, F_NAR - F_OUT))

    n_fill = E_PAD - N_EDGES
    trash = N_NODES + jnp.arange(n_fill, dtype=jnp.int32) % (N_PAD - N_NODES)
    src_e = jnp.concatenate([edge_index[0], trash]).reshape(E_PAD // CHUNK, CHUNK)
    dst_e = jnp.concatenate([edge_index[1], trash]).reshape(E_PAD // CHUNK, CHUNK)
    src_off = jnp.stack([src_e, src_e + N_PAD])
    features_p = jnp.pad(features, ((0, N_PAD - N_NODES), (0, 0)))

    do_parts, di_parts = _deg(src_e, dst_e)
    deg_out = (do_parts[0] + do_parts[1]).reshape(N_PAD, 1)
    deg_in = (di_parts[0] + di_parts[1]).reshape(N_PAD, 1)

    lo, hi = _tc_first(features_p, deg_out, W1)
    parts = _agg_split(jnp.concatenate([lo, hi]), src_off, dst_e)
    lo, hi = _tc_mid(parts, deg_in, b1.reshape(1, -1), deg_out, W2, F_HID, True)
    parts = _agg_split(jnp.concatenate([lo, hi]), src_off, dst_e)
    h = _tc_mid(parts, deg_in, b2.reshape(1, -1), deg_out, W3p, F_NAR, False)
    parts = _agg_nar(h, src_e, dst_e)
    out = _tc_final(parts, deg_in, b3p.reshape(1, -1))
    return out[:N_NODES, :F_OUT]


# speedup vs baseline: 4.5384x; 1.1295x over previous
import functools

import jax
import jax.numpy as jnp
from jax import lax
from jax.experimental import pallas as pl
from jax.experimental.pallas import tpu as pltpu
from jax.experimental.pallas import tpu_sc as plsc

N_NODES = 10000
N_EDGES = 320000
F_IN = 128
F_HID = 128
F_HALF = 64
F_OUT = 40
F_NAR = 48

NC = 2
NS = 16
NW = NC * NS
CHUNK = 128
TPC = 80
NTPC = 160
E_PAD = NW * TPC * CHUNK
NCHUNK = 128
NODE_SLOTS = 5
N_PAD = NODE_SLOTS * NS * NCHUNK
NB = 4

ROW_BLK = 1024


def _sc_mesh():
    return plsc.VectorSubcoreMesh(
        core_axis_name="c", subcore_axis_name="s", num_cores=NC, num_subcores=NS
    )


def _fill_zero_2d(buf, rows, d):
    z = jnp.zeros((16,), jnp.float32)

    def row(i, carry):
        for j in range(d // 16):
            buf[i, pl.ds(j * 16, 16)] = z
        return carry

    lax.fori_loop(0, rows, row, 0)


@functools.partial(
    pl.kernel,
    out_type=jax.ShapeDtypeStruct((NC, N_PAD, F_HALF), jnp.float32),
    mesh=_sc_mesh(),
    compiler_params=pltpu.CompilerParams(use_tc_tiling_on_sc=False),
    scratch_types=(
        [pltpu.VMEM((NTPC, CHUNK), jnp.int32)] * 2
        + [pltpu.VMEM((CHUNK, F_HALF), jnp.float32)] * NB
        + [pltpu.SemaphoreType.DMA] * (2 * NB)
        + [pltpu.VMEM_SHARED((N_PAD, F_HALF), jnp.float32)]
    ),
)
def _agg_split(table, src_off, dst_e, out, sidx, didx,
               g0, g1, g2, g3, gs0, gs1, gs2, gs3, ss0, ss1, ss2, ss3, acc):
    gbufs = (g0, g1, g2, g3)
    gsems = (gs0, gs1, gs2, gs3)
    ssems = (ss0, ss1, ss2, ss3)
    c = lax.axis_index("c")
    s = lax.axis_index("s")

    pltpu.sync_copy(src_off.at[c, pl.ds(s * NTPC, NTPC)], sidx)
    pltpu.sync_copy(dst_e.at[pl.ds(s * NTPC, NTPC)], didx)

    _fill_zero_2d(gbufs[0], NCHUNK, F_HALF)

    def zero_chunk(k, carry):
        t = s + NS * k
        pltpu.sync_copy(gbufs[0], acc.at[pl.ds(t * NCHUNK, NCHUNK)])
        return carry

    lax.fori_loop(0, NODE_SLOTS, zero_chunk, 0)
    plsc.subcore_barrier()

    def gat(k, b):
        return pltpu.make_async_copy(table.at[sidx.at[k]], gbufs[b], gsems[b])

    def sca(k, b):
        return pltpu.make_async_copy(gbufs[b], acc.at[didx.at[k]], ssems[b])

    for b in range(NB):
        gat(b, b).start()

    def group(g, carry):
        k0 = g * NB
        for b in range(NB):
            gat(k0 + b, b).wait()
            sca(k0 + b, b).start(add=True)
        for b in range(NB):
            sca(k0 + b, b).wait()

            @pl.when(k0 + NB + b < NTPC)
            def _():
                gat(k0 + NB + b, b).start()

        return carry

    lax.fori_loop(0, NTPC // NB, group, 0)
    plsc.subcore_barrier()

    def wb_chunk(k, carry):
        r0 = (s + NS * k) * NCHUNK
        pltpu.sync_copy(acc.at[pl.ds(r0, NCHUNK)], gbufs[0])
        pltpu.sync_copy(gbufs[0], out.at[c, pl.ds(r0, NCHUNK)])
        return carry

    lax.fori_loop(0, NODE_SLOTS, wb_chunk, 0)


@functools.partial(
    pl.kernel,
    out_type=jax.ShapeDtypeStruct((NC, N_PAD, F_NAR), jnp.float32),
    mesh=_sc_mesh(),
    compiler_params=pltpu.CompilerParams(use_tc_tiling_on_sc=False),
    scratch_types=(
        [pltpu.VMEM((TPC, CHUNK), jnp.int32)] * 2
        + [pltpu.VMEM((CHUNK, F_NAR), jnp.float32)] * NB
        + [pltpu.SemaphoreType.DMA] * (2 * NB)
        + [pltpu.VMEM_SHARED((N_PAD, F_NAR), jnp.float32)]
    ),
)
def _agg_nar(table, src_e, dst_e, out, sidx, didx,
             g0, g1, g2, g3, gs0, gs1, gs2, gs3, ss0, ss1, ss2, ss3, acc):
    gbufs = (g0, g1, g2, g3)
    gsems = (gs0, gs1, gs2, gs3)
    ssems = (ss0, ss1, ss2, ss3)
    c = lax.axis_index("c")
    s = lax.axis_index("s")
    tid = c * NS + s
    pltpu.sync_copy(src_e.at[pl.ds(tid * TPC, TPC)], sidx)
    pltpu.sync_copy(dst_e.at[pl.ds(tid * TPC, TPC)], didx)

    _fill_zero_2d(gbufs[0], NCHUNK, F_NAR)

    def zero_chunk(k, carry):
        t = s + NS * k
        pltpu.sync_copy(gbufs[0], acc.at[pl.ds(t * NCHUNK, NCHUNK)])
        return carry

    lax.fori_loop(0, NODE_SLOTS, zero_chunk, 0)
    plsc.subcore_barrier()

    def gat(k, b):
        return pltpu.make_async_copy(table.at[sidx.at[k]], gbufs[b], gsems[b])

    def sca(k, b):
        return pltpu.make_async_copy(gbufs[b], acc.at[didx.at[k]], ssems[b])

    for b in range(NB):
        gat(b, b).start()

    def group(g, carry):
        k0 = g * NB
        for b in range(NB):
            gat(k0 + b, b).wait()
            sca(k0 + b, b).start(add=True)
        for b in range(NB):
            sca(k0 + b, b).wait()

            @pl.when(k0 + NB + b < TPC)
            def _():
                gat(k0 + NB + b, b).start()

        return carry

    lax.fori_loop(0, TPC // NB, group, 0)
    plsc.subcore_barrier()

    def wb_chunk(k, carry):
        r0 = (s + NS * k) * NCHUNK
        pltpu.sync_copy(acc.at[pl.ds(r0, NCHUNK)], gbufs[0])
        pltpu.sync_copy(gbufs[0], out.at[c, pl.ds(r0, NCHUNK)])
        return carry

    lax.fori_loop(0, NODE_SLOTS, wb_chunk, 0)


@functools.partial(
    pl.kernel,
    out_type=(
        jax.ShapeDtypeStruct((NC, N_PAD), jnp.float32),
        jax.ShapeDtypeStruct((NC, N_PAD), jnp.float32),
    ),
    mesh=_sc_mesh(),
    scratch_types=[
        pltpu.VMEM((TPC, CHUNK), jnp.int32),
        pltpu.VMEM((TPC, CHUNK), jnp.int32),
        pltpu.VMEM((CHUNK,), jnp.float32),
        pltpu.VMEM((CHUNK,), jnp.float32),
        pltpu.SemaphoreType.DMA,
        pltpu.SemaphoreType.DMA,
        pltpu.VMEM_SHARED((N_PAD,), jnp.float32),
        pltpu.VMEM_SHARED((N_PAD,), jnp.float32),
    ],
)
def _deg(src_e, dst_e, out_o, out_i, sidx, didx, ones, buf, sem_o, sem_i, acc_o, acc_i):
    c = lax.axis_index("c")
    s = lax.axis_index("s")
    tid = c * NS + s
    one = jnp.ones((16,), jnp.float32)
    z = jnp.zeros((16,), jnp.float32)
    for j in range(CHUNK // 16):
        ones[pl.ds(j * 16, 16)] = one
        buf[pl.ds(j * 16, 16)] = z

    pltpu.sync_copy(src_e.at[pl.ds(tid * TPC, TPC)], sidx)
    pltpu.sync_copy(dst_e.at[pl.ds(tid * TPC, TPC)], didx)

    def zero_chunk(k, carry):
        t = s + NS * k
        pltpu.sync_copy(buf, acc_o.at[pl.ds(t * NCHUNK, NCHUNK)])
        pltpu.sync_copy(buf, acc_i.at[pl.ds(t * NCHUNK, NCHUNK)])
        return carry

    lax.fori_loop(0, NODE_SLOTS, zero_chunk, 0)
    plsc.subcore_barrier()

    def fire(k, carry):
        pltpu.make_async_copy(ones, acc_o.at[sidx.at[k]], sem_o).start(add=True)
        pltpu.make_async_copy(ones, acc_i.at[didx.at[k]], sem_i).start(add=True)
        return carry

    lax.fori_loop(0, TPC, fire, 0)

    def drain(k, carry):
        pltpu.make_async_copy(ones, acc_o.at[sidx.at[0]], sem_o).wait()
        pltpu.make_async_copy(ones, acc_i.at[didx.at[0]], sem_i).wait()
        return carry

    lax.fori_loop(0, TPC, drain, 0)
    plsc.subcore_barrier()

    def wb_chunk(k, carry):
        r0 = (s + NS * k) * NCHUNK
        pltpu.sync_copy(acc_o.at[pl.ds(r0, NCHUNK)], buf)
        pltpu.sync_copy(buf, out_o.at[c, pl.ds(r0, NCHUNK)])
        pltpu.sync_copy(acc_i.at[pl.ds(r0, NCHUNK)], buf)
        pltpu.sync_copy(buf, out_i.at[c, pl.ds(r0, NCHUNK)])
        return carry

    lax.fori_loop(0, NODE_SLOTS, wb_chunk, 0)


def _norm(deg):
    return jnp.where(deg > 0, lax.rsqrt(jnp.maximum(deg, 1.0)), 0.0)


def _tc_first(features, deg_out, W):
    def body(x_ref, d_ref, w_ref, lo_ref, hi_ref):
        ns = _norm(d_ref[...])
        h = jnp.dot(x_ref[...] * ns, w_ref[...], preferred_element_type=jnp.float32)
        lo_ref[...] = h[:, :F_HALF]
        hi_ref[...] = h[:, F_HALF:]

    return pl.pallas_call(
        body,
        grid=(N_PAD // ROW_BLK,),
        in_specs=[
            pl.BlockSpec((ROW_BLK, F_IN), lambda i: (i, 0)),
            pl.BlockSpec((ROW_BLK, 1), lambda i: (i, 0)),
            pl.BlockSpec((F_IN, F_HID), lambda i: (0, 0)),
        ],
        out_specs=[pl.BlockSpec((ROW_BLK, F_HALF), lambda i: (i, 0))] * 2,
        out_shape=[jax.ShapeDtypeStruct((N_PAD, F_HALF), jnp.float32)] * 2,
    )(features, deg_out, W)


def _tc_mid(parts, deg_in, b, deg_out, W, d_out, split_out):
    def body(p_ref, di_ref, b_ref, do_ref, w_ref, *o_refs):
        nd = _norm(di_ref[...])
        ns = _norm(do_ref[...])
        p = jnp.concatenate([p_ref[0], p_ref[1]], axis=1)
        h = jnp.maximum(p * nd + b_ref[...], 0.0)
        r = jnp.dot(h * ns, w_ref[...], preferred_element_type=jnp.float32)
        if split_out:
            o_refs[0][...] = r[:, :F_HALF]
            o_refs[1][...] = r[:, F_HALF:]
        else:
            o_refs[0][...] = r

    if split_out:
        out_specs = [pl.BlockSpec((ROW_BLK, F_HALF), lambda i: (i, 0))] * 2
        out_shape = [jax.ShapeDtypeStruct((N_PAD, F_HALF), jnp.float32)] * 2
    else:
        out_specs = pl.BlockSpec((ROW_BLK, d_out), lambda i: (i, 0))
        out_shape = jax.ShapeDtypeStruct((N_PAD, d_out), jnp.float32)

    return pl.pallas_call(
        body,
        grid=(N_PAD // ROW_BLK,),
        in_specs=[
            pl.BlockSpec((NC, ROW_BLK, F_HALF), lambda i: (0, i, 0)),
            pl.BlockSpec((ROW_BLK, 1), lambda i: (i, 0)),
            pl.BlockSpec((1, F_HID), lambda i: (0, 0)),
            pl.BlockSpec((ROW_BLK, 1), lambda i: (i, 0)),
            pl.BlockSpec((F_HID, d_out), lambda i: (0, 0)),
        ],
        out_specs=out_specs,
        out_shape=out_shape,
    )(parts, deg_in, b, deg_out, W)


def _tc_final(parts, deg_in, b):
    def body(p_ref, di_ref, b_ref, o_ref):
        nd = _norm(di_ref[...])
        o_ref[...] = (p_ref[0] + p_ref[1]) * nd + b_ref[...]

    return pl.pallas_call(
        body,
        grid=(N_PAD // ROW_BLK,),
        in_specs=[
            pl.BlockSpec((NC, ROW_BLK, F_NAR), lambda i: (0, i, 0)),
            pl.BlockSpec((ROW_BLK, 1), lambda i: (i, 0)),
            pl.BlockSpec((1, F_NAR), lambda i: (0, 0)),
        ],
        out_specs=pl.BlockSpec((ROW_BLK, F_NAR), lambda i: (i, 0)),
        out_shape=jax.ShapeDtypeStruct((N_PAD, F_NAR), jnp.float32),
    )(parts, deg_in, b)


def kernel(features, edge_index, W1, b1, W2, b2, W3, b3):
    W3p = jnp.pad(W3, ((0, 0), (0, F_NAR - F_OUT)))
    b3p = jnp.pad(b3, (0, F_NAR - F_OUT))

    n_fill = E_PAD - N_EDGES
    trash = N_NODES + jnp.arange(n_fill, dtype=jnp.int32) % (N_PAD - N_NODES)
    src_e = jnp.concatenate([edge_index[0], trash]).reshape(E_PAD // CHUNK, CHUNK)
    dst_e = jnp.concatenate([edge_index[1], trash]).reshape(E_PAD // CHUNK, CHUNK)
    src_off = jnp.stack([src_e, src_e + N_PAD])
    features_p = jnp.pad(features, ((0, N_PAD - N_NODES), (0, 0)))

    do_parts, di_parts = _deg(src_e, dst_e)
    deg_out = (do_parts[0] + do_parts[1]).reshape(N_PAD, 1)
    deg_in = (di_parts[0] + di_parts[1]).reshape(N_PAD, 1)

    lo, hi = _tc_first(features_p, deg_out, W1)
    parts = _agg_split(jnp.concatenate([lo, hi]), src_off, dst_e)
    lo, hi = _tc_mid(parts, deg_in, b1.reshape(1, -1), deg_out, W2, F_HID, True)
    parts = _agg_split(jnp.concatenate([lo, hi]), src_off, dst_e)
    h = _tc_mid(parts, deg_in, b2.reshape(1, -1), deg_out, W3p, F_NAR, False)
    parts = _agg_nar(h, src_e, dst_e)
    out = _tc_final(parts, deg_in, b3p.reshape(1, -1))
    return out[:N_NODES, :F_OUT]


# speedup vs baseline: 4.6002x; 1.0136x over previous
import functools

import jax
import jax.numpy as jnp
from jax import lax
from jax.experimental import pallas as pl
from jax.experimental.pallas import tpu as pltpu
from jax.experimental.pallas import tpu_sc as plsc

N_NODES = 10000
N_EDGES = 320000
F_IN = 128
F_HID = 128
F_HALF = 64
F_OUT = 40
F_NAR = 48

NC = 2
NS = 16
NW = NC * NS
CHUNK = 128
TPC = 80
NTPC = 160
E_PAD = NW * TPC * CHUNK
NCHUNK = 128
NODE_SLOTS = 5
N_PAD = NODE_SLOTS * NS * NCHUNK
NB = 5

ROW_BLK = 1024


def _sc_mesh():
    return plsc.VectorSubcoreMesh(
        core_axis_name="c", subcore_axis_name="s", num_cores=NC, num_subcores=NS
    )


def _fill_zero_2d(buf, rows, d):
    z = jnp.zeros((16,), jnp.float32)

    def row(i, carry):
        for j in range(d // 16):
            buf[i, pl.ds(j * 16, 16)] = z
        return carry

    lax.fori_loop(0, rows, row, 0)


@functools.partial(
    pl.kernel,
    out_type=jax.ShapeDtypeStruct((NC, N_PAD, F_HALF), jnp.float32),
    mesh=_sc_mesh(),
    compiler_params=pltpu.CompilerParams(use_tc_tiling_on_sc=False),
    scratch_types=(
        [pltpu.VMEM((NTPC, CHUNK), jnp.int32)] * 2
        + [pltpu.VMEM((CHUNK, F_HALF), jnp.float32)] * NB
        + [pltpu.SemaphoreType.DMA] * (2 * NB)
        + [pltpu.VMEM_SHARED((N_PAD, F_HALF), jnp.float32)]
    ),
)
def _agg_split(table, src_off, dst_e, out, sidx, didx,
               g0, g1, g2, g3, g4,
               gs0, gs1, gs2, gs3, gs4,
               ss0, ss1, ss2, ss3, ss4, acc):
    gbufs = (g0, g1, g2, g3, g4)
    gsems = (gs0, gs1, gs2, gs3, gs4)
    ssems = (ss0, ss1, ss2, ss3, ss4)
    c = lax.axis_index("c")
    s = lax.axis_index("s")

    pltpu.sync_copy(src_off.at[c, pl.ds(s * NTPC, NTPC)], sidx)
    pltpu.sync_copy(dst_e.at[pl.ds(s * NTPC, NTPC)], didx)

    _fill_zero_2d(gbufs[0], NCHUNK, F_HALF)

    def zero_chunk(k, carry):
        t = s + NS * k
        pltpu.sync_copy(gbufs[0], acc.at[pl.ds(t * NCHUNK, NCHUNK)])
        return carry

    lax.fori_loop(0, NODE_SLOTS, zero_chunk, 0)
    plsc.subcore_barrier()

    def gat(k, b):
        return pltpu.make_async_copy(table.at[sidx.at[k]], gbufs[b], gsems[b])

    def sca(k, b):
        return pltpu.make_async_copy(gbufs[b], acc.at[didx.at[k]], ssems[b])

    for b in range(NB):
        gat(b, b).start()

    def group(g, carry):
        k0 = g * NB
        for b in range(NB):
            gat(k0 + b, b).wait()
            sca(k0 + b, b).start(add=True)
        for b in range(NB):
            sca(k0 + b, b).wait()

            @pl.when(k0 + NB + b < NTPC)
            def _():
                gat(k0 + NB + b, b).start()

        return carry

    lax.fori_loop(0, NTPC // NB, group, 0)
    plsc.subcore_barrier()

    def wb_chunk(k, carry):
        r0 = (s + NS * k) * NCHUNK
        pltpu.sync_copy(acc.at[pl.ds(r0, NCHUNK)], gbufs[0])
        pltpu.sync_copy(gbufs[0], out.at[c, pl.ds(r0, NCHUNK)])
        return carry

    lax.fori_loop(0, NODE_SLOTS, wb_chunk, 0)


@functools.partial(
    pl.kernel,
    out_type=jax.ShapeDtypeStruct((NC, N_PAD, F_NAR), jnp.float32),
    mesh=_sc_mesh(),
    compiler_params=pltpu.CompilerParams(use_tc_tiling_on_sc=False),
    scratch_types=(
        [pltpu.VMEM((TPC, CHUNK), jnp.int32)] * 2
        + [pltpu.VMEM((CHUNK, F_NAR), jnp.float32)] * NB
        + [pltpu.SemaphoreType.DMA] * (2 * NB)
        + [pltpu.VMEM_SHARED((N_PAD, F_NAR), jnp.float32)]
    ),
)
def _agg_nar(table, src_e, dst_e, out, sidx, didx,
             g0, g1, g2, g3, g4,
             gs0, gs1, gs2, gs3, gs4,
             ss0, ss1, ss2, ss3, ss4, acc):
    gbufs = (g0, g1, g2, g3, g4)
    gsems = (gs0, gs1, gs2, gs3, gs4)
    ssems = (ss0, ss1, ss2, ss3, ss4)
    c = lax.axis_index("c")
    s = lax.axis_index("s")
    tid = c * NS + s
    pltpu.sync_copy(src_e.at[pl.ds(tid * TPC, TPC)], sidx)
    pltpu.sync_copy(dst_e.at[pl.ds(tid * TPC, TPC)], didx)

    _fill_zero_2d(gbufs[0], NCHUNK, F_NAR)

    def zero_chunk(k, carry):
        t = s + NS * k
        pltpu.sync_copy(gbufs[0], acc.at[pl.ds(t * NCHUNK, NCHUNK)])
        return carry

    lax.fori_loop(0, NODE_SLOTS, zero_chunk, 0)
    plsc.subcore_barrier()

    def gat(k, b):
        return pltpu.make_async_copy(table.at[sidx.at[k]], gbufs[b], gsems[b])

    def sca(k, b):
        return pltpu.make_async_copy(gbufs[b], acc.at[didx.at[k]], ssems[b])

    for b in range(NB):
        gat(b, b).start()

    def group(g, carry):
        k0 = g * NB
        for b in range(NB):
            gat(k0 + b, b).wait()
            sca(k0 + b, b).start(add=True)
        for b in range(NB):
            sca(k0 + b, b).wait()

            @pl.when(k0 + NB + b < TPC)
            def _():
                gat(k0 + NB + b, b).start()

        return carry

    lax.fori_loop(0, TPC // NB, group, 0)
    plsc.subcore_barrier()

    def wb_chunk(k, carry):
        r0 = (s + NS * k) * NCHUNK
        pltpu.sync_copy(acc.at[pl.ds(r0, NCHUNK)], gbufs[0])
        pltpu.sync_copy(gbufs[0], out.at[c, pl.ds(r0, NCHUNK)])
        return carry

    lax.fori_loop(0, NODE_SLOTS, wb_chunk, 0)


@functools.partial(
    pl.kernel,
    out_type=(
        jax.ShapeDtypeStruct((NC, N_PAD), jnp.float32),
        jax.ShapeDtypeStruct((NC, N_PAD), jnp.float32),
    ),
    mesh=_sc_mesh(),
    scratch_types=[
        pltpu.VMEM((TPC, CHUNK), jnp.int32),
        pltpu.VMEM((TPC, CHUNK), jnp.int32),
        pltpu.VMEM((CHUNK,), jnp.float32),
        pltpu.VMEM((CHUNK,), jnp.float32),
        pltpu.SemaphoreType.DMA,
        pltpu.SemaphoreType.DMA,
        pltpu.VMEM_SHARED((N_PAD,), jnp.float32),
        pltpu.VMEM_SHARED((N_PAD,), jnp.float32),
    ],
)
def _deg(src_e, dst_e, out_o, out_i, sidx, didx, ones, buf, sem_o, sem_i, acc_o, acc_i):
    c = lax.axis_index("c")
    s = lax.axis_index("s")
    tid = c * NS + s
    one = jnp.ones((16,), jnp.float32)
    z = jnp.zeros((16,), jnp.float32)
    for j in range(CHUNK // 16):
        ones[pl.ds(j * 16, 16)] = one
        buf[pl.ds(j * 16, 16)] = z

    pltpu.sync_copy(src_e.at[pl.ds(tid * TPC, TPC)], sidx)
    pltpu.sync_copy(dst_e.at[pl.ds(tid * TPC, TPC)], didx)

    def zero_chunk(k, carry):
        t = s + NS * k
        pltpu.sync_copy(buf, acc_o.at[pl.ds(t * NCHUNK, NCHUNK)])
        pltpu.sync_copy(buf, acc_i.at[pl.ds(t * NCHUNK, NCHUNK)])
        return carry

    lax.fori_loop(0, NODE_SLOTS, zero_chunk, 0)
    plsc.subcore_barrier()

    def fire(k, carry):
        pltpu.make_async_copy(ones, acc_o.at[sidx.at[k]], sem_o).start(add=True)
        pltpu.make_async_copy(ones, acc_i.at[didx.at[k]], sem_i).start(add=True)
        return carry

    lax.fori_loop(0, TPC, fire, 0)

    def drain(k, carry):
        pltpu.make_async_copy(ones, acc_o.at[sidx.at[0]], sem_o).wait()
        pltpu.make_async_copy(ones, acc_i.at[didx.at[0]], sem_i).wait()
        return carry

    lax.fori_loop(0, TPC, drain, 0)
    plsc.subcore_barrier()

    def wb_chunk(k, carry):
        r0 = (s + NS * k) * NCHUNK
        pltpu.sync_copy(acc_o.at[pl.ds(r0, NCHUNK)], buf)
        pltpu.sync_copy(buf, out_o.at[c, pl.ds(r0, NCHUNK)])
        pltpu.sync_copy(acc_i.at[pl.ds(r0, NCHUNK)], buf)
        pltpu.sync_copy(buf, out_i.at[c, pl.ds(r0, NCHUNK)])
        return carry

    lax.fori_loop(0, NODE_SLOTS, wb_chunk, 0)


def _norm(deg):
    return jnp.where(deg > 0, lax.rsqrt(jnp.maximum(deg, 1.0)), 0.0)


def _tc_first(features, deg_out, W):
    def body(x_ref, d_ref, w_ref, lo_ref, hi_ref):
        ns = _norm(d_ref[...])
        h = jnp.dot(x_ref[...] * ns, w_ref[...], preferred_element_type=jnp.float32)
        lo_ref[...] = h[:, :F_HALF]
        hi_ref[...] = h[:, F_HALF:]

    return pl.pallas_call(
        body,
        grid=(N_PAD // ROW_BLK,),
        in_specs=[
            pl.BlockSpec((ROW_BLK, F_IN), lambda i: (i, 0)),
            pl.BlockSpec((ROW_BLK, 1), lambda i: (i, 0)),
            pl.BlockSpec((F_IN, F_HID), lambda i: (0, 0)),
        ],
        out_specs=[pl.BlockSpec((ROW_BLK, F_HALF), lambda i: (i, 0))] * 2,
        out_shape=[jax.ShapeDtypeStruct((N_PAD, F_HALF), jnp.float32)] * 2,
    )(features, deg_out, W)


def _tc_mid(parts, deg_in, b, deg_out, W, d_out, split_out):
    def body(p_ref, di_ref, b_ref, do_ref, w_ref, *o_refs):
        nd = _norm(di_ref[...])
        ns = _norm(do_ref[...])
        p = jnp.concatenate([p_ref[0], p_ref[1]], axis=1)
        h = jnp.maximum(p * nd + b_ref[...], 0.0)
        r = jnp.dot(h * ns, w_ref[...], preferred_element_type=jnp.float32)
        if split_out:
            o_refs[0][...] = r[:, :F_HALF]
            o_refs[1][...] = r[:, F_HALF:]
        else:
            o_refs[0][...] = r

    if split_out:
        out_specs = [pl.BlockSpec((ROW_BLK, F_HALF), lambda i: (i, 0))] * 2
        out_shape = [jax.ShapeDtypeStruct((N_PAD, F_HALF), jnp.float32)] * 2
    else:
        out_specs = pl.BlockSpec((ROW_BLK, d_out), lambda i: (i, 0))
        out_shape = jax.ShapeDtypeStruct((N_PAD, d_out), jnp.float32)

    return pl.pallas_call(
        body,
        grid=(N_PAD // ROW_BLK,),
        in_specs=[
            pl.BlockSpec((NC, ROW_BLK, F_HALF), lambda i: (0, i, 0)),
            pl.BlockSpec((ROW_BLK, 1), lambda i: (i, 0)),
            pl.BlockSpec((1, F_HID), lambda i: (0, 0)),
            pl.BlockSpec((ROW_BLK, 1), lambda i: (i, 0)),
            pl.BlockSpec((F_HID, d_out), lambda i: (0, 0)),
        ],
        out_specs=out_specs,
        out_shape=out_shape,
    )(parts, deg_in, b, deg_out, W)


def _tc_final(parts, deg_in, b):
    def body(p_ref, di_ref, b_ref, o_ref):
        nd = _norm(di_ref[...])
        o_ref[...] = (p_ref[0] + p_ref[1]) * nd + b_ref[...]

    return pl.pallas_call(
        body,
        grid=(N_PAD // ROW_BLK,),
        in_specs=[
            pl.BlockSpec((NC, ROW_BLK, F_NAR), lambda i: (0, i, 0)),
            pl.BlockSpec((ROW_BLK, 1), lambda i: (i, 0)),
            pl.BlockSpec((1, F_NAR), lambda i: (0, 0)),
        ],
        out_specs=pl.BlockSpec((ROW_BLK, F_NAR), lambda i: (i, 0)),
        out_shape=jax.ShapeDtypeStruct((N_PAD, F_NAR), jnp.float32),
    )(parts, deg_in, b)


def kernel(features, edge_index, W1, b1, W2, b2, W3, b3):
    W3p = jnp.pad(W3, ((0, 0), (0, F_NAR - F_OUT)))
    b3p = jnp.pad(b3, (0, F_NAR - F_OUT))

    n_fill = E_PAD - N_EDGES
    trash = N_NODES + jnp.arange(n_fill, dtype=jnp.int32) % (N_PAD - N_NODES)
    src_e = jnp.concatenate([edge_index[0], trash]).reshape(E_PAD // CHUNK, CHUNK)
    dst_e = jnp.concatenate([edge_index[1], trash]).reshape(E_PAD // CHUNK, CHUNK)
    src_off = jnp.stack([src_e, src_e + N_PAD])
    features_p = jnp.pad(features, ((0, N_PAD - N_NODES), (0, 0)))

    do_parts, di_parts = _deg(src_e, dst_e)
    deg_out = (do_parts[0] + do_parts[1]).reshape(N_PAD, 1)
    deg_in = (di_parts[0] + di_parts[1]).reshape(N_PAD, 1)

    lo, hi = _tc_first(features_p, deg_out, W1)
    parts = _agg_split(jnp.concatenate([lo, hi]), src_off, dst_e)
    lo, hi = _tc_mid(parts, deg_in, b1.reshape(1, -1), deg_out, W2, F_HID, True)
    parts = _agg_split(jnp.concatenate([lo, hi]), src_off, dst_e)
    h = _tc_mid(parts, deg_in, b2.reshape(1, -1), deg_out, W3p, F_NAR, False)
    parts = _agg_nar(h, src_e, dst_e)
    out = _tc_final(parts, deg_in, b3p.reshape(1, -1))
    return out[:N_NODES, :F_OUT]
